# baseline (speedup 1.0000x reference)
import functools

import jax
import jax.numpy as jnp
from jax import lax
from jax.experimental import pallas as pl
from jax.experimental.pallas import tpu as pltpu
from jax.experimental.pallas import tpu_sc as plsc

N_ROWS = 1000000
D = 32
N_IDX = 524288

NC = 2
NS = 16
NW = NC * NS

B_PER_W = N_IDX // NW
CHUNK = 1024
NCHUNKS = B_PER_W // CHUNK

_mesh = plsc.VectorSubcoreMesh(core_axis_name="c", subcore_axis_name="s")


@functools.partial(
    pl.kernel,
    mesh=_mesh,
    out_type=jax.ShapeDtypeStruct((2 * N_IDX, D), jnp.float32),
    scratch_types=[
        pltpu.VMEM((CHUNK,), jnp.int32),
        pltpu.VMEM((CHUNK,), jnp.int32),
        pltpu.VMEM((CHUNK, D), jnp.float32),
        pltpu.VMEM((CHUNK, D), jnp.float32),
        pltpu.SemaphoreType.DMA,
        pltpu.SemaphoreType.DMA,
        pltpu.SemaphoreType.DMA,
        pltpu.SemaphoreType.DMA,
    ],
    compiler_params=pltpu.CompilerParams(use_tc_tiling_on_sc=False),
)
def _gather_concat(feat0_hbm, feat1_hbm, idx0_hbm, idx1_hbm, out_hbm,
                   idx_v0, idx_v1, rows_v0, rows_v1, gs0, gs1, ss0, ss1):
    wid = lax.axis_index("s") * NC + lax.axis_index("c")
    base = wid * B_PER_W

    idx_bufs = (idx_v0, idx_v1)
    row_bufs = (rows_v0, rows_v1)
    gsems = (gs0, gs1)
    ssems = (ss0, ss1)

    chunks = []
    for tbl, idxh, obase in ((feat0_hbm, idx0_hbm, 0),
                             (feat1_hbm, idx1_hbm, N_IDX)):
        for j in range(NCHUNKS):
            chunks.append((tbl, idxh, obase, j))

    def out_slice(obase, j):
        return out_hbm.at[pl.ds(obase + base + j * CHUNK, CHUNK)]

    pend_gather = [None, None]
    pend_store = [None, None]
    for k, (tbl, idxh, obase, j) in enumerate(chunks):
        b = k & 1
        if pend_store[b] is not None:
            pend_store[b].wait()
            pend_store[b] = None
        pltpu.sync_copy(idxh.at[pl.ds(base + j * CHUNK, CHUNK)], idx_bufs[b])
        pend_gather[b] = pltpu.async_copy(tbl.at[idx_bufs[b]], row_bufs[b],
                                          gsems[b])
        ob = 1 - b
        if pend_gather[ob] is not None:
            pend_gather[ob].wait()
            pend_gather[ob] = None
            _, _, pobase, pj = chunks[k - 1]
            pend_store[ob] = pltpu.async_copy(row_bufs[ob],
                                              out_slice(pobase, pj), ssems[ob])

    b = (len(chunks) - 1) & 1
    pend_gather[b].wait()
    _, _, obase, j = chunks[-1]
    pend_store[b] = pltpu.async_copy(row_bufs[b], out_slice(obase, j), ssems[b])
    for b in (0, 1):
        if pend_store[b] is not None:
            pend_store[b].wait()


def kernel(feat_0, feat_1, index_map_0, index_map_1):
    return _gather_concat(feat_0, feat_1,
                          index_map_0.astype(jnp.int32),
                          index_map_1.astype(jnp.int32))
